# baseline (speedup 1.0000x reference)
import functools

import jax
import jax.numpy as jnp
from jax import lax
from jax.experimental import pallas as pl
from jax.experimental.pallas import tpu as pltpu
from jax.experimental.pallas import tpu_sc as plsc

_VOCAB = 1000000
_D = 64
_B = 4096
_T = 200
_NC = 2
_NS = 16
_NW = _NC * _NS
_BL = _B // _NW
_BLK = _D * _BL
_OST = 8 * _BL
_SCALE = 8.0

_mesh = plsc.VectorSubcoreMesh(core_axis_name="c", subcore_axis_name="s")


@functools.partial(
    pl.kernel,
    mesh=_mesh,
    out_type=jax.ShapeDtypeStruct((_T * _NW * _BLK,), jnp.float32),
    scratch_types=(
        [pltpu.VMEM((_T, _BL), jnp.int32)]
        + [pltpu.VMEM((_BL, _D), jnp.float32)] * 2
        + [pltpu.VMEM((_BLK,), jnp.float32)] * 2
        + [pltpu.SemaphoreType.DMA] * 4
    ),
    compiler_params=pltpu.CompilerParams(
        use_tc_tiling_on_sc=False, needs_layout_passes=False
    ),
)
def _embed(xt_hbm, table_hbm, out_hbm, idx_v, rb0, rb1, ob0, ob1,
           g0, g1, s0, s1):
    rbs, obs, gsems, ssems = (rb0, rb1), (ob0, ob1), (g0, g1), (s0, s1)
    w = lax.axis_index("s") * _NC + lax.axis_index("c")

    pltpu.sync_copy(xt_hbm.at[:, pl.ds(w * _BL, _BL)], idx_v)

    def gather_desc(t, p):
        return pltpu.make_async_copy(
            table_hbm.at[idx_v.at[t]], rbs[p], gsems[p]
        )

    def store_descs(t, p):
        base = t * (_NW * _BLK) + w * _OST
        return [
            pltpu.make_async_copy(
                obs[p].at[pl.ds(o * _OST, _OST)],
                out_hbm.at[pl.ds(base + o * (_NW * _OST), _OST)],
                ssems[p],
            )
            for o in range(_D // 8)
        ]

    def start_store(t, p):
        for d in store_descs(t, p):
            d.start()

    def wait_store(t, p):
        for d in store_descs(t, p):
            d.wait()

    lane = lax.iota(jnp.int32, 16)
    addr = [(lane + 16 * k) * _BL for k in range(_D // 16)]

    def transpose_scale(p):
        rb, ob = rbs[p], obs[p]

        def body(b, carry):
            for k in range(_D // 16):
                v = rb[b, pl.ds(16 * k, 16)] * _SCALE
                plsc.store_scatter(ob, [addr[k] + b], v)
            return carry

        lax.fori_loop(0, _BL, body, 0, unroll=4)

    def unit(t, p):
        gather_desc(t, p).wait()
        transpose_scale(p)
        start_store(t, p)

    gather_desc(0, 0).start()
    gather_desc(1, 1).start()
    unit(0, 0)
    gather_desc(2, 0).start()
    unit(1, 1)
    gather_desc(3, 1).start()

    def step(i, carry):
        t = i * 2
        for p in range(2):
            wait_store(t + p - 2, p)
            unit(t + p, p)
            gather_desc(t + p + 2, p).start()
        return carry

    lax.fori_loop(1, _T // 2 - 1, step, 0, unroll=False)

    wait_store(_T - 4, 0)
    unit(_T - 2, 0)
    wait_store(_T - 3, 1)
    unit(_T - 1, 1)
    wait_store(_T - 2, 0)
    wait_store(_T - 1, 1)


def kernel(x, table):
    xt = jnp.transpose(x)
    flat = _embed(xt, table)
    out5 = flat.reshape(_T, _D // 8, _B // _BL, 8, _BL)
    out = jnp.transpose(out5, (2, 4, 0, 1, 3))
    return out.reshape(_B, _T, _D)


# speedup vs baseline: 1.0052x; 1.0052x over previous
import functools

import jax
import jax.numpy as jnp
from jax import lax
from jax.experimental import pallas as pl
from jax.experimental.pallas import tpu as pltpu
from jax.experimental.pallas import tpu_sc as plsc

_VOCAB = 1000000
_D = 64
_B = 4096
_T = 200
_NC = 2
_NS = 16
_NW = _NC * _NS
_BL = _B // _NW
_BLK = _D * _BL
_OST = 8 * _BL
_SCALE = 8.0

_mesh = plsc.VectorSubcoreMesh(core_axis_name="c", subcore_axis_name="s")


@functools.partial(
    pl.kernel,
    mesh=_mesh,
    out_type=jax.ShapeDtypeStruct((_T, _D // 8, _NW, 8, _BL), jnp.float32),
    scratch_types=(
        [pltpu.VMEM((_T, _BL), jnp.int32)]
        + [pltpu.VMEM((_BL, _D), jnp.float32)] * 2
        + [pltpu.VMEM((_D // 8, 8, _BL), jnp.float32)] * 2
        + [pltpu.SemaphoreType.DMA] * 4
    ),
    compiler_params=pltpu.CompilerParams(
        use_tc_tiling_on_sc=False, needs_layout_passes=False
    ),
)
def _embed(xt_hbm, table_hbm, out_hbm, idx_v, rb0, rb1, ob0, ob1,
           g0, g1, s0, s1):
    rbs, obs, gsems, ssems = (rb0, rb1), (ob0, ob1), (g0, g1), (s0, s1)
    w = lax.axis_index("s") * _NC + lax.axis_index("c")

    pltpu.sync_copy(xt_hbm.at[:, pl.ds(w * _BL, _BL)], idx_v)

    def gather_desc(t, p):
        return pltpu.make_async_copy(
            table_hbm.at[idx_v.at[t]], rbs[p], gsems[p]
        )

    def store_desc(t, p):
        return pltpu.make_async_copy(obs[p], out_hbm.at[t, :, w], ssems[p])

    def start_store(t, p):
        store_desc(t, p).start()

    def wait_store(t, p):
        store_desc(t, p).wait()

    lane = lax.iota(jnp.int32, 16)
    o_ids = [lax.shift_right_logical(lane + 16 * k, 3) for k in range(_D // 16)]
    ci_ids = [lax.bitwise_and(lane + 16 * k, 7) for k in range(_D // 16)]

    def transpose_scale(p):
        rb, ob = rbs[p], obs[p]

        def body(b, carry):
            col = lane * 0 + b
            for k in range(_D // 16):
                v = rb[b, pl.ds(16 * k, 16)] * _SCALE
                plsc.store_scatter(ob, [o_ids[k], ci_ids[k], col], v)
            return carry

        lax.fori_loop(0, _BL, body, 0, unroll=4)

    def unit(t, p):
        gather_desc(t, p).wait()
        transpose_scale(p)
        start_store(t, p)

    gather_desc(0, 0).start()
    gather_desc(1, 1).start()
    unit(0, 0)
    gather_desc(2, 0).start()
    unit(1, 1)
    gather_desc(3, 1).start()

    def step(i, carry):
        t = i * 2
        for p in range(2):
            wait_store(t + p - 2, p)
            unit(t + p, p)
            gather_desc(t + p + 2, p).start()
        return carry

    lax.fori_loop(1, _T // 2 - 1, step, 0, unroll=False)

    wait_store(_T - 4, 0)
    unit(_T - 2, 0)
    wait_store(_T - 3, 1)
    unit(_T - 1, 1)
    wait_store(_T - 2, 0)
    wait_store(_T - 1, 1)


def kernel(x, table):
    xt = jnp.transpose(x)
    out5 = _embed(xt, table)
    out = jnp.transpose(out5, (2, 4, 0, 1, 3))
    return out.reshape(_B, _T, _D)


# speedup vs baseline: 1.8574x; 1.8478x over previous
import functools

import jax
import jax.numpy as jnp
from jax import lax
from jax.experimental import pallas as pl
from jax.experimental.pallas import tpu as pltpu
from jax.experimental.pallas import tpu_sc as plsc

_VOCAB = 1000000
_D = 64
_B = 4096
_T = 200
_N = _B * _T
_NC = 2
_NS = 16
_NW = _NC * _NS
_PER_W = _N // _NW
_CH = 320
_NCH = _PER_W // _CH
_NB = 4
_SCALE = 8.0

_mesh = plsc.VectorSubcoreMesh(core_axis_name="c", subcore_axis_name="s")


@functools.partial(
    pl.kernel,
    mesh=_mesh,
    out_type=jax.ShapeDtypeStruct((_N, 2 * _D), jnp.float32),
    scratch_types=(
        [pltpu.VMEM((_PER_W,), jnp.int32)]
        + [pltpu.VMEM((_CH, _D), jnp.float32)] * _NB
        + [pltpu.SemaphoreType.DMA] * (2 * _NB)
    ),
    compiler_params=pltpu.CompilerParams(
        use_tc_tiling_on_sc=False, needs_layout_passes=False
    ),
)
def _embed(idx_hbm, table_hbm, out_hbm, idx_v, *scratch):
    bufs = scratch[:_NB]
    gsems = scratch[_NB:2 * _NB]
    ssems = scratch[2 * _NB:]

    wid = lax.axis_index("s") * _NC + lax.axis_index("c")
    base = wid * _PER_W
    pltpu.sync_copy(idx_hbm.at[pl.ds(base, _PER_W)], idx_v)

    def gather_desc(c, b):
        src = table_hbm.at[idx_v.at[pl.ds(c * _CH, _CH)]]
        return pltpu.make_async_copy(src, bufs[b], gsems[b])

    def store_desc(c, b):
        dst = out_hbm.at[pl.ds(base + c * _CH, _CH), pl.ds(0, _D)]
        return pltpu.make_async_copy(bufs[b], dst, ssems[b])

    def scale(b):
        buf = bufs[b]

        def row(r, carry):
            for k in range(_D // 16):
                sl = (r, pl.ds(16 * k, 16))
                buf[sl] = buf[sl] * _SCALE
            return carry

        lax.fori_loop(0, _CH, row, 0, unroll=8)

    def head(c, b):
        gather_desc(c, b).wait()
        scale(b)
        store_desc(c, b).start()

    def tail(c, b):
        nb = (b + 2) % _NB
        store_desc(c - 2, nb).wait()
        gather_desc(c + 2, nb).start()

    gather_desc(0, 0).start()
    gather_desc(1, 1).start()
    head(0, 0)
    gather_desc(2, 2).start()
    head(1, 1)
    gather_desc(3, 3).start()
    head(2, 2)
    tail(2, 2)
    head(3, 3)
    tail(3, 3)

    def step(o, carry):
        c0 = o * _NB
        for u in range(_NB):
            head(c0 + u, u)
            tail(c0 + u, u)
        return carry

    lax.fori_loop(1, _NCH // _NB - 1, step, 0, unroll=False)

    head(_NCH - 4, 0)
    tail(_NCH - 4, 0)
    head(_NCH - 3, 1)
    tail(_NCH - 3, 1)
    head(_NCH - 2, 2)
    head(_NCH - 1, 3)
    store_desc(_NCH - 4, 0).wait()
    store_desc(_NCH - 3, 1).wait()
    store_desc(_NCH - 2, 2).wait()
    store_desc(_NCH - 1, 3).wait()


def kernel(x, table):
    idx = x.reshape(_N)
    outp = _embed(idx, table)
    return outp[:, :_D].reshape(_B, _T, _D)
